# baseline (speedup 1.0000x reference)
import functools

import jax
import jax.numpy as jnp
from jax import lax
from jax.experimental import pallas as pl
from jax.experimental.pallas import tpu as pltpu
from jax.experimental.pallas import tpu_sc as plsc

NC = 2
NS = 16
L = 16
NW = NC * NS
NBUF = 4


def _chunks(S):
    c0 = (min(S, 128) // 8) * 8
    out = []
    off = 0
    while off < S:
        n = min(S - off, c0)
        out.append((off, n))
        off += n
    return out


def _body(S, H, rows_per_w, seq_hbm, seg_hbm, emb_hbm, posbf_hbm, type_hbm,
          out_hbm, *scratch):
    idx_v = scratch[0:NBUF]
    segv = scratch[NBUF:2 * NBUF]
    rows_v = scratch[2 * NBUF:3 * NBUF]
    pos_bf, type_v = scratch[3 * NBUF:3 * NBUF + 2]
    gsems = scratch[3 * NBUF + 2:3 * NBUF + 2 + NBUF]
    wsems = scratch[3 * NBUF + 2 + NBUF:3 * NBUF + 2 + 2 * NBUF]
    isems = scratch[3 * NBUF + 2 + 2 * NBUF:3 * NBUF + 2 + 3 * NBUF]

    nh = H // L
    last = rows_per_w - 1
    wid = lax.axis_index("s") * NC + lax.axis_index("c")
    base = wid * rows_per_w
    chunks = _chunks(S)

    pltpu.sync_copy(type_hbm, type_v)
    pltpu.sync_copy(posbf_hbm, pos_bf)
    t0 = [type_v[0, pl.ds(h * L, L)] for h in range(nh)]
    t1 = [type_v[1, pl.ds(h * L, L)] for h in range(nh)]
    one = jnp.full((L,), 1, jnp.int32)

    def idx_fetch(j, row):
        gb = base + row
        pltpu.async_copy(seq_hbm.at[gb], idx_v[j], isems[j])
        pltpu.async_copy(seg_hbm.at[gb], segv[j], isems[j])

    def idx_wait(j):
        pltpu.make_async_copy(seq_hbm.at[base], idx_v[j], isems[j]).wait()
        pltpu.make_async_copy(seg_hbm.at[base], segv[j], isems[j]).wait()

    def issue_gather(j):
        for (o, n) in chunks:
            pltpu.async_copy(
                emb_hbm.at[idx_v[j].at[pl.ds(o, n)]],
                rows_v[j].at[pl.ds(o, n)], gsems[j])

    def wait_gather(j):
        for (o, n) in chunks:
            pltpu.make_async_copy(
                emb_hbm.at[idx_v[j].at[pl.ds(o, n)]],
                rows_v[j].at[pl.ds(o, n)], gsems[j]).wait()

    def start_write(j, row):
        pltpu.async_copy(rows_v[j], out_hbm.at[base + row], wsems[j])

    def wait_write(j):
        pltpu.make_async_copy(rows_v[j], out_hbm.at[base], wsems[j]).wait()

    def add_pos(rv, segq, s, i):
        lane = jnp.full((L,), i, jnp.int32)
        m = jnp.take_along_axis(segq, lane, 0,
                                mode="promise_in_bounds") == one
        for h2 in range(nh // 2):
            pw = plsc.bitcast(pos_bf[s, pl.ds(h2 * L, L)], jnp.bfloat16)
            pab = plsc.unpack(pw, format=plsc.PackFormat.INTERLEAVED)
            for t in range(2):
                h = h2 * 2 + t
                tadd = jnp.where(m, t1[h], t0[h])
                r = rv[s, pl.ds(h * L, L)]
                rv[s, pl.ds(h * L, L)] = (r + pab[t]) + tadd

    def compute(k):
        rv = rows_v[k % NBUF]
        sv = segv[k % NBUF]
        nfull = (S // L) * L

        @plsc.parallel_loop(0, nfull, step=L)
        def blk(s0):
            segq = sv[pl.ds(s0, L)]

            @plsc.parallel_loop(0, L, unroll=4)
            def pos_body(i):
                add_pos(rv, segq, s0 + i, i)

        if S % L:
            segq = sv[pl.ds(S - L, L)]
            for i in range(L - S % L, L):
                add_pos(rv, segq, S - L + i, i)

    def slot(j, row, wwait, issue2, fetch4):
        if issue2:
            if wwait:
                wait_write((j + 2) % NBUF)
            idx_wait((j + 2) % NBUF)
            issue_gather((j + 2) % NBUF)
        wait_gather(j)
        compute(j)
        start_write(j, row)
        if fetch4:
            idx_fetch(j, row + NBUF)

    for k in range(NBUF):
        idx_fetch(k, k)
    idx_wait(0)
    issue_gather(0)
    idx_wait(1)
    issue_gather(1)
    for k in range(NBUF):
        slot(k, k, wwait=(k >= 2), issue2=True, fetch4=True)

    def steady(g, c):
        for j in range(NBUF):
            slot(j, NBUF * g + j, wwait=True, issue2=True, fetch4=True)
        return c

    n_steady = (rows_per_w - 2 * NBUF) // NBUF
    lax.fori_loop(1, 1 + n_steady, steady, 0)

    for k in range(rows_per_w - NBUF, rows_per_w):
        slot(k % NBUF, k, wwait=True, issue2=(k + 2 <= last), fetch4=False)
    for k in range(NBUF):
        wait_write(k)


def kernel(in_seq, in_seg, word_embeddings, positional_encoding,
           type_embeddings):
    B, S = in_seq.shape
    H = word_embeddings.shape[1]
    assert B % NW == 0
    rows_per_w = B // NW
    assert rows_per_w % NBUF == 0 and rows_per_w >= 2 * NBUF

    seq = in_seq.astype(jnp.int32)
    seg = in_seg.astype(jnp.int32)

    pos = positional_encoding[:S].astype(jnp.float32)
    pos3 = pos.reshape(S, H // (2 * L), 2, L)
    a16 = pos3[:, :, 0, :].astype(jnp.bfloat16)
    b16 = pos3[:, :, 1, :].astype(jnp.bfloat16)
    inter = jnp.stack([a16, b16], axis=-1)
    posbf = jax.lax.bitcast_convert_type(
        inter.reshape(S, H // (2 * L), L, 2), jnp.int32).reshape(S, H // 2)

    mesh = plsc.VectorSubcoreMesh(core_axis_name="c", subcore_axis_name="s")
    f = pl.kernel(
        functools.partial(_body, S, H, rows_per_w),
        out_type=jax.ShapeDtypeStruct((B, S, H), jnp.float32),
        mesh=mesh,
        compiler_params=pltpu.CompilerParams(needs_layout_passes=False),
        scratch_types=(
            [pltpu.VMEM((S,), jnp.int32) for _ in range(NBUF)]
            + [pltpu.VMEM((S,), jnp.int32) for _ in range(NBUF)]
            + [pltpu.VMEM((S, H), jnp.float32) for _ in range(NBUF)]
            + [pltpu.VMEM((S, H // 2), jnp.int32),
               pltpu.VMEM((2, H), jnp.float32)]
            + [pltpu.SemaphoreType.DMA for _ in range(3 * NBUF)]
        ),
    )
    return f(seq, seg, word_embeddings, posbf, type_embeddings)
